# baseline (speedup 1.0000x reference)
import functools

import jax
import jax.numpy as jnp
from jax import lax
from jax.experimental import pallas as pl
from jax.experimental.pallas import tpu as pltpu
from jax.experimental.pallas import tpu_sc as plsc

N_NODES = 100000
N_EDGES = 6400000
D_MAX_SQ = 16.0
EPS = 1e-8

_info = plsc.get_sparse_core_info()
_NC = _info.num_cores
_NS = _info.num_subcores
_L = _info.num_lanes
_NW = _NC * _NS
_EPW = N_EDGES // _NW
_CHUNK = 1600
_MW = _CHUNK // 4
_NCHUNKS = _EPW // _CHUNK
_NPAIRS = (_NCHUNKS - 1) // 2
_GV = _CHUNK // (4 * _L)


def _rsqrt(x):
    k = plsc.bitcast(x, jnp.int32)
    r = plsc.bitcast(jnp.int32(0x5F3759DF) - (k >> 1), jnp.float32)
    for _ in range(2):
        r = r * (1.5 - 0.5 * x * r * r)
    return r


_mesh = plsc.VectorSubcoreMesh(core_axis_name="c", subcore_axis_name="s")


@functools.partial(
    pl.kernel,
    mesh=_mesh,
    compiler_params=pltpu.CompilerParams(
        needs_layout_passes=False, use_tc_tiling_on_sc=False),
    out_type=jax.ShapeDtypeStruct((_NW, 2, _L), jnp.float32),
    scratch_types=[
        [pltpu.VMEM((_CHUNK,), jnp.int32)] * 2,
        [pltpu.VMEM((_CHUNK,), jnp.int32)] * 2,
        [pltpu.VMEM((_MW,), jnp.int32)] * 2,
        [pltpu.VMEM((_CHUNK, 8), jnp.float32)] * 2,
        [pltpu.VMEM((_CHUNK, 8), jnp.float32)] * 2,
        pltpu.VMEM((2, _L), jnp.float32),
        pltpu.VMEM_SHARED((N_NODES, 8), jnp.float32),
        [pltpu.SemaphoreType.DMA] * 2,
        [pltpu.SemaphoreType.DMA] * 2,
        [pltpu.SemaphoreType.DMA] * 2,
        [pltpu.SemaphoreType.DMA] * 2,
    ],
)
def _edge_loss_sc(table, srci, dsti, maskw, out,
                  sidx, didx, m, srows, drows, acc_v, spt,
                  sem_li, sem_lm, sem_gs, sem_gd):
    sid = lax.axis_index("s")
    wid = sid * _NC + lax.axis_index("c")
    base0 = wid * _EPW
    mbase0 = wid * (_EPW // 4)
    iota = lax.iota(jnp.int32, _L)
    cols = [jnp.full((_L,), j, jnp.int32) for j in range(7)]

    rpt = N_NODES // _NS
    pltpu.sync_copy(table.at[pl.ds(sid * rpt, rpt)],
                    spt.at[pl.ds(sid * rpt, rpt)])
    plsc.subcore_barrier()

    def lin_idx(g, p):
        base = base0 + g * _CHUNK
        return (pltpu.make_async_copy(srci.at[pl.ds(base, _CHUNK)],
                                      sidx[p], sem_li[p]),
                pltpu.make_async_copy(dsti.at[pl.ds(base, _CHUNK)],
                                      didx[p], sem_li[p]))

    def lin_mask(g, p):
        base = mbase0 + g * _MW
        return pltpu.make_async_copy(maskw.at[pl.ds(base, _MW)],
                                     m[p], sem_lm[p])

    def gathers(p):
        return (pltpu.make_async_copy(spt.at[sidx[p]], srows[p], sem_gs[p]),
                pltpu.make_async_copy(spt.at[didx[p]], drows[p], sem_gd[p]))

    def compute(p, accs):
        sr, dr, mm = srows[p], drows[p], m[p]

        def body(i, accs):
            mw = mm[pl.ds(i * _L, _L)]
            rb = (i * _L + iota) * 4
            out = []
            for k in range(4):
                num2, cnt2 = accs[k]
                row = rb + k
                s = [plsc.load_gather(sr, [row, cols[j]]) for j in range(7)]
                d = [plsc.load_gather(dr, [row, cols[j]]) for j in range(6)]
                mbit = (mw >> (8 * k)) & 1
                t0 = s[0] - d[0]
                t1 = s[1] - d[1]
                t2 = s[2] - d[2]
                a = jnp.maximum(t0 * t0 + t1 * t1 + t2 * t2, EPS)
                g0 = s[3] - d[3]
                g1 = s[4] - d[4]
                g2 = s[5] - d[5]
                b = jnp.maximum(g0 * g0 + g1 * g1 + g2 * g2, EPS)
                ab = a * b
                sq = ab * _rsqrt(ab)
                sqerr = (a + b) - 2.0 * sq
                keep = jnp.where(a < D_MAX_SQ, mbit, 0).astype(jnp.float32)
                out.append((num2 + keep * s[6] * sqerr, cnt2 + keep))
            return tuple(out)

        return plsc.parallel_loop(0, _GV, 1, carry=accs)(body)

    for cp in lin_idx(0, 0) + lin_idx(1, 1):
        cp.start()
    lin_mask(0, 0).start()
    lin_mask(1, 1).start()
    for cp in lin_idx(0, 0):
        cp.wait()
    for cp in gathers(0):
        cp.start()

    def pair_body(g2, accs):
        for b in (0, 1):
            g = g2 * 2 + b
            p, q = b, 1 - b
            more = g2 < _NPAIRS - 1

            for cp in lin_idx(g + 1, q):
                cp.wait()
            for cp in gathers(q):
                cp.start()

            for cp in gathers(p):
                cp.wait()

            def prefetch_idx():
                for cp in lin_idx(g + 2, p):
                    cp.start()

            if b == 0:
                prefetch_idx()
            else:
                pl.when(more)(prefetch_idx)
            lin_mask(g, p).wait()
            accs = compute(p, accs)
            if b == 0:
                lin_mask(g + 2, p).start()
            else:
                pl.when(more)(lambda: lin_mask(g + 2, p).start())
        return accs

    zero = jnp.zeros((_L,), jnp.float32)
    accs = lax.fori_loop(0, _NPAIRS, pair_body,
                         tuple((zero, zero) for _ in range(4)))
    last = _NCHUNKS - 1
    for cp in gathers(0):
        cp.wait()
    lin_mask(last, 0).wait()
    accs = compute(0, accs)

    num = accs[0][0] + accs[1][0] + (accs[2][0] + accs[3][0])
    cnt = accs[0][1] + accs[1][1] + (accs[2][1] + accs[3][1])
    acc_v[0, :] = num
    acc_v[1, :] = cnt
    pltpu.sync_copy(acc_v, out.at[wid])


def kernel(x_1_true, lig_x, src_idxs, dst_idxs, lig_ue_mask,
           node_batch_idxs_lig, time_weights):
    tw_node = time_weights[node_batch_idxs_lig]
    table = jnp.concatenate(
        [x_1_true, lig_x, tw_node[:, None],
         jnp.zeros((N_NODES, 1), jnp.float32)], axis=1)
    maskw = lax.bitcast_convert_type(
        lig_ue_mask.astype(jnp.uint8).reshape(N_EDGES // 4, 4), jnp.int32)
    parts = _edge_loss_sc(table,
                          src_idxs.astype(jnp.int32),
                          dst_idxs.astype(jnp.int32),
                          maskw)
    num = jnp.sum(parts[:, 0, :])
    cnt = jnp.sum(parts[:, 1, :])
    return num / jnp.maximum(cnt, 1.0)


# speedup vs baseline: 1.6202x; 1.6202x over previous
import functools

import jax
import jax.numpy as jnp
from jax import lax
from jax.experimental import pallas as pl
from jax.experimental.pallas import tpu as pltpu
from jax.experimental.pallas import tpu_sc as plsc

N_NODES = 100000
N_EDGES = 6400000
D_MAX_SQ = 16.0
EPS = 1e-8

_info = plsc.get_sparse_core_info()
_NC = _info.num_cores
_NS = _info.num_subcores
_L = _info.num_lanes
_NW = _NC * _NS
_EPW = N_EDGES // _NW
_CHUNK = 1600
_MW = _CHUNK // 4
_NCHUNKS = _EPW // _CHUNK
_NPAIRS = (_NCHUNKS - 1) // 2
_GV = _CHUNK // (4 * _L)


def _rsqrt(x):
    k = plsc.bitcast(x, jnp.int32)
    r = plsc.bitcast(jnp.int32(0x5F3759DF) - (k >> 1), jnp.float32)
    for _ in range(2):
        r = r * (1.5 - 0.5 * x * r * r)
    return r


_mesh = plsc.VectorSubcoreMesh(core_axis_name="c", subcore_axis_name="s")


@functools.partial(
    pl.kernel,
    mesh=_mesh,
    compiler_params=pltpu.CompilerParams(
        needs_layout_passes=False, use_tc_tiling_on_sc=False),
    out_type=jax.ShapeDtypeStruct((_NW, 2, _L), jnp.float32),
    scratch_types=[
        [pltpu.VMEM((_CHUNK,), jnp.int32)] * 2,
        [pltpu.VMEM((_CHUNK,), jnp.int32)] * 2,
        [pltpu.VMEM((_MW,), jnp.int32)] * 2,
        [pltpu.VMEM((_CHUNK, 8), jnp.float32)] * 2,
        [pltpu.VMEM((_CHUNK, 8), jnp.float32)] * 2,
        pltpu.VMEM((2, _L), jnp.float32),
        pltpu.VMEM_SHARED((N_NODES, 8), jnp.float32),
        [pltpu.SemaphoreType.DMA] * 2,
        [pltpu.SemaphoreType.DMA] * 2,
        [pltpu.SemaphoreType.DMA] * 2,
        [pltpu.SemaphoreType.DMA] * 2,
    ],
)
def _edge_loss_sc(table, srci, dsti, maskw, out,
                  sidx, didx, m, srows, drows, acc_v, spt,
                  sem_li, sem_lm, sem_gs, sem_gd):
    sid = lax.axis_index("s")
    wid = sid * _NC + lax.axis_index("c")
    base0 = wid * _EPW
    mbase0 = wid * (_EPW // 4)
    iota = lax.iota(jnp.int32, _L)
    cols = [jnp.full((_L,), j, jnp.int32) for j in range(7)]
    dgidx = iota >> 2
    bshift = (iota & 3) * 8

    rpt = N_NODES // _NS
    pltpu.sync_copy(table.at[pl.ds(sid * rpt, rpt)],
                    spt.at[pl.ds(sid * rpt, rpt)])
    plsc.subcore_barrier()

    def lin_idx(g, p):
        base = base0 + g * _CHUNK
        return (pltpu.make_async_copy(srci.at[pl.ds(base, _CHUNK)],
                                      sidx[p], sem_li[p]),
                pltpu.make_async_copy(dsti.at[pl.ds(base, _CHUNK)],
                                      didx[p], sem_li[p]))

    def lin_mask(g, p):
        base = mbase0 + g * _MW
        return pltpu.make_async_copy(maskw.at[pl.ds(base, _MW)],
                                     m[p], sem_lm[p])

    def gathers(p):
        return (pltpu.make_async_copy(spt.at[sidx[p]], srows[p], sem_gs[p]),
                pltpu.make_async_copy(spt.at[didx[p]], drows[p], sem_gd[p]))

    def compute(p, accs):
        sr, dr, mm = srows[p], drows[p], m[p]

        def body(i, accs):
            mw = mm[pl.ds(i * _L, _L)]
            rb = i * (4 * _L) + iota
            out = []
            for k in range(4):
                num2, cnt2 = accs[k]
                row = rb + k * _L
                s = [plsc.load_gather(sr, [row, cols[j]]) for j in range(7)]
                d = [plsc.load_gather(dr, [row, cols[j]]) for j in range(6)]
                mword = lax.gather(
                    mw, (dgidx + 4 * k)[:, None],
                    lax.GatherDimensionNumbers(
                        offset_dims=(), collapsed_slice_dims=(0,),
                        start_index_map=(0,)),
                    slice_sizes=(1,),
                    mode=lax.GatherScatterMode.PROMISE_IN_BOUNDS)
                mbit = (mword >> bshift) & 1
                t0 = s[0] - d[0]
                t1 = s[1] - d[1]
                t2 = s[2] - d[2]
                a = jnp.maximum(t0 * t0 + t1 * t1 + t2 * t2, EPS)
                g0 = s[3] - d[3]
                g1 = s[4] - d[4]
                g2 = s[5] - d[5]
                b = jnp.maximum(g0 * g0 + g1 * g1 + g2 * g2, EPS)
                ab = a * b
                sq = ab * _rsqrt(ab)
                sqerr = (a + b) - 2.0 * sq
                keep = jnp.where(a < D_MAX_SQ, mbit, 0).astype(jnp.float32)
                out.append((num2 + keep * s[6] * sqerr, cnt2 + keep))
            return tuple(out)

        return plsc.parallel_loop(0, _GV, 1, carry=accs)(body)

    for cp in lin_idx(0, 0) + lin_idx(1, 1):
        cp.start()
    lin_mask(0, 0).start()
    lin_mask(1, 1).start()
    for cp in lin_idx(0, 0):
        cp.wait()
    for cp in gathers(0):
        cp.start()

    def pair_body(g2, accs):
        for b in (0, 1):
            g = g2 * 2 + b
            p, q = b, 1 - b
            more = g2 < _NPAIRS - 1

            for cp in lin_idx(g + 1, q):
                cp.wait()
            for cp in gathers(q):
                cp.start()

            for cp in gathers(p):
                cp.wait()

            def prefetch_idx():
                for cp in lin_idx(g + 2, p):
                    cp.start()

            if b == 0:
                prefetch_idx()
            else:
                pl.when(more)(prefetch_idx)
            lin_mask(g, p).wait()
            accs = compute(p, accs)
            if b == 0:
                lin_mask(g + 2, p).start()
            else:
                pl.when(more)(lambda: lin_mask(g + 2, p).start())
        return accs

    zero = jnp.zeros((_L,), jnp.float32)
    accs = lax.fori_loop(0, _NPAIRS, pair_body,
                         tuple((zero, zero) for _ in range(4)))
    last = _NCHUNKS - 1
    for cp in gathers(0):
        cp.wait()
    lin_mask(last, 0).wait()
    accs = compute(0, accs)

    num = accs[0][0] + accs[1][0] + (accs[2][0] + accs[3][0])
    cnt = accs[0][1] + accs[1][1] + (accs[2][1] + accs[3][1])
    acc_v[0, :] = num
    acc_v[1, :] = cnt
    pltpu.sync_copy(acc_v, out.at[wid])


def kernel(x_1_true, lig_x, src_idxs, dst_idxs, lig_ue_mask,
           node_batch_idxs_lig, time_weights):
    tw_node = time_weights[node_batch_idxs_lig]
    table = jnp.concatenate(
        [x_1_true, lig_x, tw_node[:, None],
         jnp.zeros((N_NODES, 1), jnp.float32)], axis=1)
    maskw = lax.bitcast_convert_type(
        lig_ue_mask.astype(jnp.uint8).reshape(N_EDGES // 4, 4), jnp.int32)
    parts = _edge_loss_sc(table,
                          src_idxs.astype(jnp.int32),
                          dst_idxs.astype(jnp.int32),
                          maskw)
    num = jnp.sum(parts[:, 0, :])
    cnt = jnp.sum(parts[:, 1, :])
    return num / jnp.maximum(cnt, 1.0)


# speedup vs baseline: 5.8919x; 3.6364x over previous
import functools

import jax
import jax.numpy as jnp
from jax import lax
from jax.experimental import pallas as pl
from jax.experimental.pallas import tpu as pltpu
from jax.experimental.pallas import tpu_sc as plsc

N_NODES = 100000
N_EDGES = 6400000
D_MAX_SQ = 16.0
EPS = 1e-8

_info = plsc.get_sparse_core_info()
_NC = _info.num_cores
_NS = _info.num_subcores
_L = _info.num_lanes
_NW = _NC * _NS
_EPW = N_EDGES // _NW
_CHUNK = 2000
_NCHUNKS = _EPW // _CHUNK
_NPAIRS = _NCHUNKS // 2
_CV = _CHUNK // _L


def _rsqrt(x):
    k = plsc.bitcast(x, jnp.int32)
    r = plsc.bitcast(jnp.int32(0x5F3759DF) - (k >> 1), jnp.float32)
    for _ in range(2):
        r = r * (1.5 - 0.5 * x * r * r)
    return r


_mesh = plsc.VectorSubcoreMesh(core_axis_name="c", subcore_axis_name="s")


@functools.partial(
    pl.kernel,
    mesh=_mesh,
    compiler_params=pltpu.CompilerParams(
        needs_layout_passes=False, use_tc_tiling_on_sc=False),
    out_type=jax.ShapeDtypeStruct((_NW, 2, _L), jnp.float32),
    scratch_types=[
        [pltpu.VMEM((_CHUNK,), jnp.int32)] * 2,
        [pltpu.VMEM((_CHUNK,), jnp.int32)] * 2,
        [pltpu.VMEM((_CHUNK,), jnp.float32)] * 2,
        [pltpu.VMEM((_CHUNK, 8), jnp.float32)] * 2,
        [pltpu.VMEM((_CHUNK, 8), jnp.float32)] * 2,
        pltpu.VMEM((2, _L), jnp.float32),
        pltpu.VMEM_SHARED((N_NODES, 8), jnp.float32),
        [pltpu.SemaphoreType.DMA] * 2,
        [pltpu.SemaphoreType.DMA] * 2,
        [pltpu.SemaphoreType.DMA] * 2,
        [pltpu.SemaphoreType.DMA] * 2,
    ],
)
def _edge_loss_sc(table, srci, dsti, maskf, out,
                  sidx, didx, m, srows, drows, acc_v, spt,
                  sem_li, sem_lm, sem_gs, sem_gd):
    sid = lax.axis_index("s")
    wid = sid * _NC + lax.axis_index("c")
    base0 = wid * _EPW
    iota = lax.iota(jnp.int32, _L)
    cols = [jnp.full((_L,), j, jnp.int32) for j in range(7)]

    rpt = N_NODES // _NS
    pltpu.sync_copy(table.at[pl.ds(sid * rpt, rpt)],
                    spt.at[pl.ds(sid * rpt, rpt)])
    plsc.subcore_barrier()

    def lin_idx(g, p):
        base = base0 + g * _CHUNK
        return (pltpu.make_async_copy(srci.at[pl.ds(base, _CHUNK)],
                                      sidx[p], sem_li[p]),
                pltpu.make_async_copy(dsti.at[pl.ds(base, _CHUNK)],
                                      didx[p], sem_li[p]))

    def lin_mask(g, p):
        base = base0 + g * _CHUNK
        return pltpu.make_async_copy(maskf.at[pl.ds(base, _CHUNK)],
                                     m[p], sem_lm[p])

    def gathers(p):
        return (pltpu.make_async_copy(spt.at[sidx[p]], srows[p], sem_gs[p]),
                pltpu.make_async_copy(spt.at[didx[p]], drows[p], sem_gd[p]))

    def compute(p, num, cnt):
        sr, dr, mm = srows[p], drows[p], m[p]

        def body(i, carry):
            num2, cnt2 = carry
            row = i * _L + iota
            s = [plsc.load_gather(sr, [row, cols[j]]) for j in range(7)]
            d = [plsc.load_gather(dr, [row, cols[j]]) for j in range(6)]
            mv = mm[pl.ds(i * _L, _L)]
            t0 = s[0] - d[0]
            t1 = s[1] - d[1]
            t2 = s[2] - d[2]
            a = jnp.maximum(t0 * t0 + t1 * t1 + t2 * t2, EPS)
            g0 = s[3] - d[3]
            g1 = s[4] - d[4]
            g2 = s[5] - d[5]
            b = jnp.maximum(g0 * g0 + g1 * g1 + g2 * g2, EPS)
            ab = a * b
            sq = ab * _rsqrt(ab)
            sqerr = (a + b) - 2.0 * sq
            keep = jnp.where(a < D_MAX_SQ, mv, 0.0)
            return (num2 + keep * s[6] * sqerr, cnt2 + keep)

        return plsc.parallel_loop(0, _CV, 1, unroll=5,
                                  carry=(num, cnt))(body)

    for cp in lin_idx(0, 0) + lin_idx(1, 1):
        cp.start()
    lin_mask(0, 0).start()
    lin_mask(1, 1).start()
    for cp in lin_idx(0, 0):
        cp.wait()
    for cp in gathers(0):
        cp.start()

    def pair_body(g2, carry):
        num, cnt = carry
        for b in (0, 1):
            g = g2 * 2 + b
            p, q = b, 1 - b
            more = g2 < _NPAIRS - 1

            def stage_next():
                for cp in lin_idx(g + 1, q):
                    cp.wait()
                for cp in gathers(q):
                    cp.start()

            if b == 0:
                stage_next()
            else:
                pl.when(more)(stage_next)

            for cp in gathers(p):
                cp.wait()

            def prefetch_idx():
                for cp in lin_idx(g + 2, p):
                    cp.start()

            pl.when(more)(prefetch_idx)
            lin_mask(g, p).wait()
            num, cnt = compute(p, num, cnt)
            pl.when(more)(lambda: lin_mask(g + 2, p).start())
        return (num, cnt)

    zero = jnp.zeros((_L,), jnp.float32)
    num, cnt = lax.fori_loop(0, _NPAIRS, pair_body, (zero, zero))
    acc_v[0, :] = num
    acc_v[1, :] = cnt
    pltpu.sync_copy(acc_v, out.at[wid])


def kernel(x_1_true, lig_x, src_idxs, dst_idxs, lig_ue_mask,
           node_batch_idxs_lig, time_weights):
    tw_node = time_weights[node_batch_idxs_lig]
    table = jnp.concatenate(
        [x_1_true, lig_x, tw_node[:, None],
         jnp.zeros((N_NODES, 1), jnp.float32)], axis=1)
    parts = _edge_loss_sc(table,
                          src_idxs.astype(jnp.int32),
                          dst_idxs.astype(jnp.int32),
                          lig_ue_mask.astype(jnp.float32))
    num = jnp.sum(parts[:, 0, :])
    cnt = jnp.sum(parts[:, 1, :])
    return num / jnp.maximum(cnt, 1.0)
